# baseline (speedup 1.0000x reference)
import jax
import jax.numpy as jnp
from jax import lax
from jax.experimental import pallas as pl
from jax.experimental.pallas import tpu as pltpu

TOKENS = 32768
DIM = 768
NUM_EXPERTS = 64
TOP_K = 2
BLOCK = 8192


def _gate_block(x_ref, w_ref, b_ref, out_ref, idx_ref):
    xb = x_ref[...]
    logits_t = lax.dot_general(
        w_ref[...], xb, (((1,), (1,)), ((), ())),
        preferred_element_type=jnp.float32,
    )
    logits_t = logits_t + b_ref[...]

    iota = lax.broadcasted_iota(jnp.int32, logits_t.shape, 0).astype(jnp.float32)
    neg_inf = jnp.float32(-jnp.inf)
    big = jnp.float32(NUM_EXPERTS)

    v1 = jnp.max(logits_t, axis=0, keepdims=True)
    i1 = jnp.min(jnp.where(logits_t == v1, iota, big), axis=0, keepdims=True)
    hit1 = iota == i1
    masked = jnp.where(hit1, neg_inf, logits_t)
    v2 = jnp.max(masked, axis=0, keepdims=True)
    i2 = jnp.min(jnp.where(masked == v2, iota, big), axis=0, keepdims=True)
    hit2 = iota == i2

    e2 = jnp.exp(v2 - v1)
    denom = 1.0 + e2
    p1 = 1.0 / denom
    p2 = e2 / denom

    out_ref[...] = jnp.where(hit1, p1, jnp.where(hit2, p2, 0.0))
    idx_ref[...] = jnp.concatenate([i1, i2], axis=0).astype(jnp.int32)


def _gate(x, W, b):
    b_col = b.reshape(NUM_EXPERTS, 1)
    grid = (TOKENS // BLOCK,)
    out_t, idx_t = pl.pallas_call(
        _gate_block,
        grid=grid,
        in_specs=[
            pl.BlockSpec((BLOCK, DIM), lambda i: (i, 0)),
            pl.BlockSpec((NUM_EXPERTS, DIM), lambda i: (0, 0)),
            pl.BlockSpec((NUM_EXPERTS, 1), lambda i: (0, 0)),
        ],
        out_specs=[
            pl.BlockSpec((NUM_EXPERTS, BLOCK), lambda i: (0, i)),
            pl.BlockSpec((TOP_K, BLOCK), lambda i: (0, i)),
        ],
        out_shape=[
            jax.ShapeDtypeStruct((NUM_EXPERTS, TOKENS), jnp.float32),
            jax.ShapeDtypeStruct((TOP_K, TOKENS), jnp.int32),
        ],
        compiler_params=pltpu.CompilerParams(
            dimension_semantics=("parallel",),
        ),
    )(x, W, b_col)
    return (out_t.T, idx_t.T)


kernel = jax.jit(_gate)


# speedup vs baseline: 1.0108x; 1.0108x over previous
import jax
import jax.numpy as jnp
from jax import lax
from jax.experimental import pallas as pl
from jax.experimental.pallas import tpu as pltpu

TOKENS = 32768
DIM = 768
NUM_EXPERTS = 64
TOP_K = 2
BLOCK = 2048


def _gate_block(x_ref, w_ref, b_ref, out_ref, idx_ref):
    xb = x_ref[...]
    logits_t = lax.dot_general(
        w_ref[...], xb, (((1,), (1,)), ((), ())),
        preferred_element_type=jnp.float32,
    )
    logits_t = logits_t + b_ref[...]

    iota = lax.broadcasted_iota(jnp.int32, logits_t.shape, 0).astype(jnp.float32)
    neg_inf = jnp.float32(-jnp.inf)
    big = jnp.float32(NUM_EXPERTS)

    v1 = jnp.max(logits_t, axis=0, keepdims=True)
    i1 = jnp.min(jnp.where(logits_t == v1, iota, big), axis=0, keepdims=True)
    hit1 = iota == i1
    masked = jnp.where(hit1, neg_inf, logits_t)
    v2 = jnp.max(masked, axis=0, keepdims=True)
    i2 = jnp.min(jnp.where(masked == v2, iota, big), axis=0, keepdims=True)
    hit2 = iota == i2

    e2 = jnp.exp(v2 - v1)
    denom = 1.0 + e2
    p1 = 1.0 / denom
    p2 = e2 / denom

    out_ref[...] = jnp.where(hit1, p1, jnp.where(hit2, p2, 0.0))
    idx_ref[...] = jnp.concatenate([i1, i2], axis=0).astype(jnp.int32)


def _gate(x, W, b):
    b_col = b.reshape(NUM_EXPERTS, 1)
    grid = (TOKENS // BLOCK,)
    out_t, idx_t = pl.pallas_call(
        _gate_block,
        grid=grid,
        in_specs=[
            pl.BlockSpec((BLOCK, DIM), lambda i: (i, 0)),
            pl.BlockSpec((NUM_EXPERTS, DIM), lambda i: (0, 0)),
            pl.BlockSpec((NUM_EXPERTS, 1), lambda i: (0, 0)),
        ],
        out_specs=[
            pl.BlockSpec((NUM_EXPERTS, BLOCK), lambda i: (0, i)),
            pl.BlockSpec((TOP_K, BLOCK), lambda i: (0, i)),
        ],
        out_shape=[
            jax.ShapeDtypeStruct((NUM_EXPERTS, TOKENS), jnp.float32),
            jax.ShapeDtypeStruct((TOP_K, TOKENS), jnp.int32),
        ],
        compiler_params=pltpu.CompilerParams(
            dimension_semantics=("parallel",),
        ),
    )(x, W, b_col)
    return (out_t.T, idx_t.T)


kernel = jax.jit(_gate)


# speedup vs baseline: 1.1005x; 1.0887x over previous
import jax
import jax.numpy as jnp
from jax import lax
from jax.experimental import pallas as pl
from jax.experimental.pallas import tpu as pltpu

TOKENS = 32768
DIM = 768
NUM_EXPERTS = 64
TOP_K = 2
BLOCK = 4096


def _gate_block(x_ref, w_ref, b_ref, out_ref, idx_ref):
    xb = x_ref[...]
    logits_t = lax.dot_general(
        w_ref[...], xb, (((1,), (1,)), ((), ())),
        preferred_element_type=jnp.float32,
    )
    logits_t = logits_t + b_ref[...].T

    iota = lax.broadcasted_iota(jnp.int32, logits_t.shape, 0).astype(jnp.float32)
    neg_inf = jnp.float32(-jnp.inf)
    big = jnp.float32(NUM_EXPERTS)

    v1 = jnp.max(logits_t, axis=0, keepdims=True)
    i1 = jnp.min(jnp.where(logits_t == v1, iota, big), axis=0, keepdims=True)
    hit1 = iota == i1
    masked = jnp.where(hit1, neg_inf, logits_t)
    v2 = jnp.max(masked, axis=0, keepdims=True)
    i2 = jnp.min(jnp.where(masked == v2, iota, big), axis=0, keepdims=True)
    hit2 = iota == i2

    e2 = jnp.exp(v2 - v1)
    denom = 1.0 + e2
    p1 = 1.0 / denom
    p2 = e2 / denom

    out_ref[...] = jnp.where(hit1, p1, jnp.where(hit2, p2, 0.0))
    idx_ref[...] = jnp.concatenate([i1, i2], axis=0).astype(jnp.int32)


def _gate(x, W, b):
    b_row = b.reshape(1, NUM_EXPERTS)
    grid = (TOKENS // BLOCK,)
    out_t, idx_t = pl.pallas_call(
        _gate_block,
        grid=grid,
        in_specs=[
            pl.BlockSpec((BLOCK, DIM), lambda i: (i, 0)),
            pl.BlockSpec((NUM_EXPERTS, DIM), lambda i: (0, 0)),
            pl.BlockSpec((1, NUM_EXPERTS), lambda i: (0, 0)),
        ],
        out_specs=[
            pl.BlockSpec((NUM_EXPERTS, BLOCK), lambda i: (0, i)),
            pl.BlockSpec((TOP_K, BLOCK), lambda i: (0, i)),
        ],
        out_shape=[
            jax.ShapeDtypeStruct((NUM_EXPERTS, TOKENS), jnp.float32),
            jax.ShapeDtypeStruct((TOP_K, TOKENS), jnp.int32),
        ],
        compiler_params=pltpu.CompilerParams(
            dimension_semantics=("parallel",),
        ),
    )(x, W, b_row)
    return (out_t.T, idx_t.T)


kernel = jax.jit(_gate)


# speedup vs baseline: 1.1037x; 1.0029x over previous
import jax
import jax.numpy as jnp
from jax import lax
from jax.experimental import pallas as pl
from jax.experimental.pallas import tpu as pltpu

TOKENS = 32768
DIM = 768
NUM_EXPERTS = 64
TOP_K = 2
BLOCK = 4096


def _gate_block(x_ref, w_ref, b_ref, out_ref, idx_ref):
    xb = x_ref[...]
    logits_t = lax.dot_general(
        w_ref[...], xb, (((1,), (1,)), ((), ())),
        preferred_element_type=jnp.float32,
    )
    logits_t = logits_t + b_ref[...].T

    iota = lax.broadcasted_iota(jnp.int32, logits_t.shape, 0).astype(jnp.float32)
    neg_inf = jnp.float32(-jnp.inf)
    big = jnp.float32(NUM_EXPERTS)

    v1 = jnp.max(logits_t, axis=0, keepdims=True)
    i1 = jnp.min(jnp.where(logits_t == v1, iota, big), axis=0, keepdims=True)
    hit1 = iota == i1
    masked = jnp.where(hit1, neg_inf, logits_t)
    v2 = jnp.max(masked, axis=0, keepdims=True)
    i2 = jnp.min(jnp.where(masked == v2, iota, big), axis=0, keepdims=True)
    hit2 = iota == i2

    e2 = jnp.exp(v2 - v1)
    denom = 1.0 + e2
    p1 = 1.0 / denom
    p2 = e2 / denom

    out_ref[...] = jnp.where(hit1, p1, jnp.where(hit2, p2, 0.0))
    idx_ref[...] = jnp.concatenate([i1, i2], axis=0).astype(jnp.int32)


def _gate(x, W, b):
    b_row = b.reshape(1, NUM_EXPERTS)
    grid = (TOKENS // BLOCK,)
    out_t, idx_t = pl.pallas_call(
        _gate_block,
        grid=grid,
        in_specs=[
            pl.BlockSpec((BLOCK, DIM), lambda i: (i, 0)),
            pl.BlockSpec((NUM_EXPERTS, DIM), lambda i: (0, 0)),
            pl.BlockSpec((1, NUM_EXPERTS), lambda i: (0, 0)),
        ],
        out_specs=[
            pl.BlockSpec((NUM_EXPERTS, BLOCK), lambda i: (0, i)),
            pl.BlockSpec((TOP_K, BLOCK), lambda i: (0, i)),
        ],
        out_shape=[
            jax.ShapeDtypeStruct((NUM_EXPERTS, TOKENS), jnp.float32),
            jax.ShapeDtypeStruct((TOP_K, TOKENS), jnp.int32),
        ],
        compiler_params=pltpu.CompilerParams(
            dimension_semantics=("arbitrary",),
        ),
    )(x, W, b_row)
    return (out_t.T, idx_t.T)


kernel = jax.jit(_gate)
